# Initial kernel scaffold; baseline (speedup 1.0000x reference)
import jax
import jax.numpy as jnp
from jax.experimental import pallas as pl


def kernel(x, weights, bias):
    raise NotImplementedError("write your pallas kernel here")


# speedup vs baseline: 1.1140x; 1.1140x over previous
import jax
import jax.numpy as jnp
from jax import lax
from jax.experimental import pallas as pl
from jax.experimental.pallas import tpu as pltpu


def _cdiv(a, b):
    return -(-a // b)


def _packed_kernel(b_ref, xp_ref, w_ref, o_ref):
    acc = lax.dot_general(
        xp_ref[...], w_ref[...],
        dimension_numbers=(((1,), (1,)), ((), ())),
        preferred_element_type=jnp.float32,
        precision=lax.Precision.HIGHEST,
    )
    o_ref[...] = (acc + b_ref[0, 0]).astype(o_ref.dtype)


def _rows_kernel(b_ref, x_ref, w_ref, o_ref):
    acc = lax.dot_general(
        x_ref[...].astype(jnp.float32), w_ref[...],
        dimension_numbers=(((1,), (1,)), ((), ())),
        preferred_element_type=jnp.float32,
        precision=lax.Precision.HIGHEST,
    )
    o_ref[...] = (acc + b_ref[0, 0]).astype(o_ref.dtype)


def _pick_tile(rows, bytes_per_row, vmem_budget=24 << 20):
    per_row_buffered = 2 * bytes_per_row
    tile = (vmem_budget // per_row_buffered) // 512 * 512
    tile = min(tile, _cdiv(rows, 512) * 512)
    return max(tile, 512)


def kernel(x, weights, bias):
    N, D = x.shape
    w_f32 = jnp.asarray(weights, jnp.float32).reshape(D)
    b_f32 = jnp.asarray(bias, jnp.float32).reshape(1, 1)
    cparams = pltpu.CompilerParams(dimension_semantics=("parallel",))
    itemsize = jnp.dtype(x.dtype).itemsize
    cost = pl.CostEstimate(flops=2 * N * D, transcendentals=0,
                           bytes_accessed=N * D * itemsize + N * itemsize)

    pack = 128 // D if (D <= 128 and 128 % D == 0) else 1
    if pack > 1 and N % pack == 0:
        G = N // pack
        xp = x.reshape(G, 128)
        lane = lax.broadcasted_iota(jnp.int32, (pack, 128), 1)
        row = lax.broadcasted_iota(jnp.int32, (pack, 128), 0)
        wp = jnp.where(lane // D == row, w_f32[lane % D], 0.0)
        tile_g = _pick_tile(G, 2 * 128 * itemsize)
        grid = _cdiv(G, tile_g)
        out = pl.pallas_call(
            _packed_kernel,
            out_shape=jax.ShapeDtypeStruct((G, pack), x.dtype),
            grid=(grid,),
            in_specs=[
                pl.BlockSpec(memory_space=pltpu.SMEM),
                pl.BlockSpec((tile_g, 128), lambda i: (i, 0)),
                pl.BlockSpec((pack, 128), lambda i: (0, 0)),
            ],
            out_specs=pl.BlockSpec((tile_g, pack), lambda i: (i, 0)),
            compiler_params=cparams,
            cost_estimate=cost,
        )(b_f32, xp, wp)
        return out.reshape(-1)

    lane_row = _cdiv(D, 128) * 128 * itemsize
    tile_n = _pick_tile(N, lane_row + 128 * itemsize)
    grid = _cdiv(N, tile_n)
    out = pl.pallas_call(
        _rows_kernel,
        out_shape=jax.ShapeDtypeStruct((N, 1), x.dtype),
        grid=(grid,),
        in_specs=[
            pl.BlockSpec(memory_space=pltpu.SMEM),
            pl.BlockSpec((tile_n, D), lambda i: (i, 0)),
            pl.BlockSpec((1, D), lambda i: (0, 0)),
        ],
        out_specs=pl.BlockSpec((tile_n, 1), lambda i: (i, 0)),
        compiler_params=cparams,
        cost_estimate=cost,
    )(b_f32, x, w_f32.reshape(1, D))
    return out.reshape(-1)
